# Initial kernel scaffold; baseline (speedup 1.0000x reference)
import jax
import jax.numpy as jnp
from jax.experimental import pallas as pl


def kernel(x, edge_index, W1, b1, W2, b2):
    raise NotImplementedError("write your pallas kernel here")


# speedup vs baseline: 22.6267x; 22.6267x over previous
import functools

import jax
import jax.numpy as jnp
from jax import lax
from jax.experimental import pallas as pl
from jax.experimental.pallas import tpu as pltpu
from jax.experimental.pallas import tpu_sc as plsc

N = 10000
E = 320000
DIM = 128
HID = 16

NC = 2
NS = 16
L = 16
NW = NC * NS

CHUNK = 128
NCHUNK = -(-E // (NW * CHUNK))
EP = NW * CHUNK * NCHUNK
EW = EP // NW

NP = 10240
RPT = NP // NS
NSUB = RPT // CHUNK

_mesh = plsc.VectorSubcoreMesh(core_axis_name="c", subcore_axis_name="s")

@functools.partial(
    pl.kernel,
    out_type=jax.ShapeDtypeStruct((NC * NP,), jnp.float32),
    mesh=_mesh,
    scratch_types=[
        pltpu.VMEM((CHUNK,), jnp.int32),
        pltpu.VMEM((CHUNK,), jnp.float32),
        pltpu.VMEM((RPT,), jnp.float32),
        pltpu.VMEM_SHARED((NP,), jnp.float32),
    ],
)
def _sc_hist(dst_hbm, out_hbm, didx, ones, stage, acc):
    c = lax.axis_index("c")
    s = lax.axis_index("s")
    wid = c * NS + s
    r0 = s * RPT
    for i in range(CHUNK // L):
        ones[pl.ds(i * L, L)] = jnp.full((L,), 1.0, jnp.float32)
    for i in range(RPT // L):
        stage[pl.ds(i * L, L)] = jnp.zeros((L,), jnp.float32)
    pltpu.sync_copy(stage, acc.at[pl.ds(r0, RPT)])
    plsc.subcore_barrier()
    base = wid * EW

    def body(j, carry):
        pltpu.sync_copy(dst_hbm.at[pl.ds(base + j * CHUNK, CHUNK)], didx)
        pltpu.sync_copy(ones, acc.at[didx], add=True)
        return carry

    lax.fori_loop(0, NCHUNK, body, 0)
    plsc.subcore_barrier()
    pltpu.sync_copy(acc.at[pl.ds(r0, RPT)], stage)
    pltpu.sync_copy(stage, out_hbm.at[pl.ds(c * NP + r0, RPT)])


@functools.partial(
    pl.kernel,
    out_type=jax.ShapeDtypeStruct((NC * NP, HID), jnp.float32),
    mesh=_mesh,
    scratch_types=[
        pltpu.VMEM((CHUNK,), jnp.int32),
        pltpu.VMEM((CHUNK,), jnp.int32),
        pltpu.VMEM((CHUNK, HID), jnp.float32),
        pltpu.VMEM_SHARED((NP, HID), jnp.float32),
        pltpu.SemaphoreType.DMA,
    ],
    compiler_params=pltpu.CompilerParams(use_tc_tiling_on_sc=False),
)
def _sc_edge16(src_hbm, dst_hbm, table_hbm, out_hbm, sidx, didx, rows, acc, sem):
    c = lax.axis_index("c")
    s = lax.axis_index("s")
    wid = c * NS + s
    r0 = s * RPT
    for i in range(CHUNK):
        rows[i, :] = jnp.zeros((L,), jnp.float32)
    for k in range(NSUB):
        pltpu.sync_copy(rows, acc.at[pl.ds(r0 + k * CHUNK, CHUNK)])
    plsc.subcore_barrier()
    base = wid * EW

    def body(j, carry):
        off = base + j * CHUNK
        pltpu.sync_copy(src_hbm.at[pl.ds(off, CHUNK)], sidx)
        pltpu.sync_copy(dst_hbm.at[pl.ds(off, CHUNK)], didx)
        pltpu.async_copy(table_hbm.at[sidx], rows, sem).wait()
        pltpu.sync_copy(rows, acc.at[didx], add=True)
        return carry

    lax.fori_loop(0, NCHUNK, body, 0)
    plsc.subcore_barrier()
    for k in range(NSUB):
        pltpu.sync_copy(acc.at[pl.ds(r0 + k * CHUNK, CHUNK)], rows)
        pltpu.sync_copy(rows, out_hbm.at[pl.ds(c * NP + r0 + k * CHUNK, CHUNK)])


@functools.partial(
    pl.kernel,
    out_type=jax.ShapeDtypeStruct((NC * NP,), jnp.float32),
    mesh=_mesh,
    scratch_types=[
        pltpu.VMEM((CHUNK,), jnp.int32),
        pltpu.VMEM((CHUNK,), jnp.int32),
        pltpu.VMEM((CHUNK,), jnp.float32),
        pltpu.VMEM((RPT,), jnp.float32),
        pltpu.VMEM_SHARED((NP,), jnp.float32),
        pltpu.SemaphoreType.DMA,
    ],
)
def _sc_edge1(src_hbm, dst_hbm, table_hbm, out_hbm,
              sidx, didx, rows, stage, acc, sem):
    c = lax.axis_index("c")
    s = lax.axis_index("s")
    wid = c * NS + s
    r0 = s * RPT
    for i in range(RPT // L):
        stage[pl.ds(i * L, L)] = jnp.zeros((L,), jnp.float32)
    pltpu.sync_copy(stage, acc.at[pl.ds(r0, RPT)])
    plsc.subcore_barrier()
    base = wid * EW

    def body(j, carry):
        off = base + j * CHUNK
        pltpu.sync_copy(src_hbm.at[pl.ds(off, CHUNK)], sidx)
        pltpu.sync_copy(dst_hbm.at[pl.ds(off, CHUNK)], didx)
        pltpu.async_copy(table_hbm.at[sidx], rows, sem).wait()
        pltpu.sync_copy(rows, acc.at[didx], add=True)
        return carry

    lax.fori_loop(0, NCHUNK, body, 0)
    plsc.subcore_barrier()
    pltpu.sync_copy(acc.at[pl.ds(r0, RPT)], stage)
    pltpu.sync_copy(stage, out_hbm.at[pl.ds(c * NP + r0, RPT)])


def _tc1_body(x_ref, w1_ref, degp_ref, g1_ref, d_ref):
    deg = degp_ref[0] + degp_ref[1] + 1.0
    d = lax.rsqrt(deg)
    d_ref[...] = d
    h = jnp.dot(x_ref[...], w1_ref[...], preferred_element_type=jnp.float32)
    g1_ref[...] = h * d


def _tc1(xp, W1, degp):
    return pl.pallas_call(
        _tc1_body,
        out_shape=[
            jax.ShapeDtypeStruct((NP, HID), jnp.float32),
            jax.ShapeDtypeStruct((NP, 1), jnp.float32),
        ],
    )(xp, W1, degp)


def _tc2_body(accp_ref, g1_ref, d_ref, b1_ref, w2_ref, g2_ref):
    a = accp_ref[0] + accp_ref[1] + g1_ref[...]
    out1 = jnp.maximum(a * d_ref[...] + b1_ref[...], 0.0)
    h2 = jnp.dot(out1, w2_ref[...], preferred_element_type=jnp.float32)
    g2_ref[...] = h2 * d_ref[...]


def _tc2(accp, g1, d, b1, W2):
    return pl.pallas_call(
        _tc2_body,
        out_shape=jax.ShapeDtypeStruct((NP, 1), jnp.float32),
    )(accp, g1, d, b1, W2)


def _tc3_body(accp_ref, g2_ref, d_ref, b2_ref, out_ref):
    a = accp_ref[0] + accp_ref[1] + g2_ref[...]
    out_ref[...] = a * d_ref[...] + b2_ref[...]


def _tc3(accp, g2, d, b2):
    return pl.pallas_call(
        _tc3_body,
        out_shape=jax.ShapeDtypeStruct((NP, 1), jnp.float32),
    )(accp, g2, d, b2)


def kernel(x, edge_index, W1, b1, W2, b2):
    src = edge_index[0]
    dst = edge_index[1]
    padi = jnp.full((EP - E,), N, dtype=jnp.int32)
    srcp = jnp.concatenate([src, padi])
    dstp = jnp.concatenate([dst, padi])
    xp = jnp.pad(x, ((0, NP - N), (0, 0)))

    degp = _sc_hist(dstp).reshape(NC, NP, 1)
    g1, d = _tc1(xp, W1, degp)
    acc1 = _sc_edge16(srcp, dstp, g1).reshape(NC, NP, HID)
    g2 = _tc2(acc1, g1, d, b1.reshape(1, HID), W2)
    acc2 = _sc_edge1(srcp, dstp, g2.reshape(NP)).reshape(NC, NP, 1)
    outp = _tc3(acc2, g2, d, b2.reshape(1, 1))
    return outp[:N, 0]
